# baseline (speedup 1.0000x reference)
import jax
import jax.numpy as jnp
from jax.experimental import pallas as pl
from jax.experimental.pallas import tpu as pltpu


def kernel(w_seq, c_seq, char_embedding, word_embedding,
           w_ih1, w_hh1, b_ih1, b_hh1, c_hx, c_cx,
           w_ih2, w_hh2, b_ih2, b_hh2, hx, cx, w_out, b_out):
    W = int(w_seq.shape[0])
    Lc = int(c_seq.shape[1])
    Hc = int(c_hx.shape[-1])
    Ew = int(word_embedding.shape[1])
    H = int(hx.shape[-1])
    T = int(w_out.shape[0])
    Cs = int(char_embedding.shape[0])
    Wp = ((W + 7) // 8) * 8
    bf = jnp.bfloat16

    cseq = c_seq
    if Wp != W:
        cseq = jnp.pad(cseq, ((0, Wp - W), (0, 0)))

    def _halve_ifo(x, hdim):
        return jnp.concatenate(
            [x[:, :2 * hdim] * 0.5, x[:, 2 * hdim:3 * hdim],
             x[:, 3 * hdim:] * 0.5], axis=1)

    def _cell(gates, c, hdim):
        t_if = jnp.tanh(gates[:, :2 * hdim])
        t_g = jnp.tanh(gates[:, 2 * hdim:3 * hdim])
        t_o = jnp.tanh(gates[:, 3 * hdim:])
        ch = 0.5 * c
        gh = 0.5 * t_g
        c2 = (t_if[:, hdim:] * ch + ch) + (t_if[:, :hdim] * gh + gh)
        th = 0.5 * jnp.tanh(c2)
        h = t_o * th + th
        return h, c2

    chunked = Lc >= 7 + (4 * H // 256 if H % 256 == 0 else 1)

    def body(wseq_ref, cseq_ref, wemb_hbm, wih2_hbm, whh2_hbm, wout_hbm,
             ce_ref, wih1_ref, whh1_ref, bih1_ref, bhh1_ref,
             ch0_ref, cc0_ref, bih2_ref, bhh2_ref, h0_ref, c0_ref, bout_ref,
             out_ref,
             wemb_scr, wih2_scr, whh2_scr, wout_scr,
             xg2_scr, whh2t_scr, ho_scr, gsem, wsem):
        for w in range(W):
            pltpu.make_async_copy(wemb_hbm.at[wseq_ref[w]],
                                  wemb_scr.at[w], gsem).start()
        pltpu.make_async_copy(wih2_hbm, wih2_scr, wsem).start()
        pltpu.make_async_copy(whh2_hbm, whh2_scr, wsem).start()
        pltpu.make_async_copy(wout_hbm, wout_scr, wsem).start()

        wih1t = wih1_ref[...].T.astype(bf)
        whh1t = _halve_ifo(whh1_ref[...].T, Hc).astype(bf)
        b1 = bih1_ref[...] + bhh1_ref[...]
        ce_projb = _halve_ifo(
            jnp.dot(ce_ref[...].astype(bf), wih1t,
                    preferred_element_type=jnp.float32) + b1,
            Hc).astype(bf)

        def _whh2t_chunk(k):
            if H % 256:
                whh2t_scr[...] = _halve_ifo(whh2_scr[...].T,
                                            H).astype(bf)
                return
            x = whh2_scr[256 * k:256 * (k + 1), :].T
            if 256 * (k + 1) <= 2 * H or 256 * k >= 3 * H:
                x = x * 0.5
            whh2t_scr[:, 256 * k:256 * (k + 1)] = x.astype(bf)

        def _word_prep(t):
            if t == 0:
                pltpu.make_async_copy(wih2_scr, wih2_scr, wsem).wait()
                pltpu.make_async_copy(whh2_scr, whh2_scr, wsem).wait()
                pltpu.make_async_copy(wout_scr, wout_scr, wsem).wait()
            if t == 1:
                prep["w2wt"] = wih2_scr[:, Hc:].T.astype(bf)
                prep["b2"] = bih2_ref[...] + bhh2_ref[...]
            if t == 2:
                pltpu.make_async_copy(wemb_scr.at[pl.ds(0, W)],
                                      wemb_scr.at[pl.ds(0, W)], gsem).wait()
                xg2_scr[...] = (jnp.dot(wemb_scr[...].astype(bf),
                                        prep["w2wt"],
                                        preferred_element_type=jnp.float32)
                                + prep["b2"])
            if t == 3:
                prep["w2ct"] = wih2_scr[:, :Hc].T.astype(bf)
            if 4 <= t < 4 + n_chunks:
                _whh2t_chunk(t - 4)
            if t == 4 + n_chunks:
                prep["woutt"] = wout_scr[...].T.astype(bf)

        prep = {}
        n_chunks = 4 * H // 256 if H % 256 == 0 else 1
        n_prep = 5 + n_chunks

        h = jnp.broadcast_to(ch0_ref[...], (Wp, Hc))
        c = jnp.broadcast_to(cc0_ref[...], (Wp, Hc))
        iota = jax.lax.broadcasted_iota(jnp.int32, (Wp, Cs), 1)
        for t in range(Lc):
            onehot = (iota == cseq_ref[:, t:t + 1]).astype(bf)
            gates = (jnp.dot(onehot, ce_projb,
                             preferred_element_type=jnp.float32)
                     + jnp.dot(h.astype(bf), whh1t,
                               preferred_element_type=jnp.float32))
            h, c = _cell(gates, c, Hc)
            if chunked and t >= 1:
                _word_prep(t - 1)
        if not chunked:
            for t in range(n_prep):
                _word_prep(t)
        elif Lc - 1 < n_prep:
            for t in range(Lc - 1, n_prep):
                _word_prep(t)

        xg2_scr[...] = _halve_ifo(
            xg2_scr[...] + jnp.dot(h.astype(bf), prep["w2ct"],
                                   preferred_element_type=jnp.float32), H)

        h2 = h0_ref[...]
        c2 = c0_ref[...]
        for w in range(W):
            gates = xg2_scr[w:w + 1, :] + jnp.dot(
                h2.astype(bf), whh2t_scr[...],
                preferred_element_type=jnp.float32)
            h2, c2 = _cell(gates, c2, H)
            ho_scr[w:w + 1, :] = h2.astype(bf)

        tag = (jnp.dot(ho_scr[...], prep["woutt"],
                       preferred_element_type=jnp.float32) + bout_ref[...])
        m = jnp.max(tag, axis=1, keepdims=True)
        z = tag - m
        lse = jnp.log(jnp.sum(jnp.exp(z), axis=1, keepdims=True))
        out_ref[...] = z - lse

    vmem_inputs = (char_embedding,
                   w_ih1, w_hh1, b_ih1.reshape(1, -1), b_hh1.reshape(1, -1),
                   c_hx.reshape(1, Hc), c_cx.reshape(1, Hc),
                   b_ih2.reshape(1, -1), b_hh2.reshape(1, -1),
                   hx.reshape(1, H), cx.reshape(1, H),
                   b_out.reshape(1, -1))

    def _full(shape):
        nd = len(shape)
        return pl.BlockSpec(shape, lambda i, _nd=nd: (0,) * _nd)

    any_spec = pl.BlockSpec(memory_space=pl.ANY)
    out = pl.pallas_call(
        body,
        out_shape=jax.ShapeDtypeStruct((Wp, T), jnp.float32),
        grid=(1,),
        in_specs=[pl.BlockSpec(memory_space=pltpu.SMEM),
                  _full(cseq.shape),
                  any_spec, any_spec, any_spec, any_spec]
                 + [_full(x.shape) for x in vmem_inputs],
        out_specs=_full((Wp, T)),
        scratch_shapes=[pltpu.VMEM((Wp, Ew), jnp.float32),
                        pltpu.VMEM(w_ih2.shape, jnp.float32),
                        pltpu.VMEM(w_hh2.shape, jnp.float32),
                        pltpu.VMEM(w_out.shape, jnp.float32),
                        pltpu.VMEM((Wp, 4 * H), jnp.float32),
                        pltpu.VMEM((H, 4 * H), bf),
                        pltpu.VMEM((Wp, H), bf),
                        pltpu.SemaphoreType.DMA,
                        pltpu.SemaphoreType.DMA],
        compiler_params=pltpu.CompilerParams(
            dimension_semantics=("arbitrary",)),
    )(w_seq, cseq, word_embedding, w_ih2, w_hh2, w_out, *vmem_inputs)
    if Wp != W:
        out = out[:W]
    return out


# speedup vs baseline: 1.0348x; 1.0348x over previous
import jax
import jax.numpy as jnp
from jax.experimental import pallas as pl
from jax.experimental.pallas import tpu as pltpu


def kernel(w_seq, c_seq, char_embedding, word_embedding,
           w_ih1, w_hh1, b_ih1, b_hh1, c_hx, c_cx,
           w_ih2, w_hh2, b_ih2, b_hh2, hx, cx, w_out, b_out):
    W = int(w_seq.shape[0])
    Lc = int(c_seq.shape[1])
    Hc = int(c_hx.shape[-1])
    Ew = int(word_embedding.shape[1])
    H = int(hx.shape[-1])
    T = int(w_out.shape[0])
    Cs = int(char_embedding.shape[0])
    Wp = ((W + 7) // 8) * 8
    bf = jnp.bfloat16

    cseq = c_seq
    if Wp != W:
        cseq = jnp.pad(cseq, ((0, Wp - W), (0, 0)))

    def _halve_ifo(x, hdim):
        return jnp.concatenate(
            [x[:, :2 * hdim] * 0.5, x[:, 2 * hdim:3 * hdim],
             x[:, 3 * hdim:] * 0.5], axis=1)

    def _cell(gates, c, hdim):
        t_if = jnp.tanh(gates[:, :2 * hdim])
        t_g = jnp.tanh(gates[:, 2 * hdim:3 * hdim])
        t_o = jnp.tanh(gates[:, 3 * hdim:])
        ch = 0.5 * c
        gh = 0.5 * t_g
        c2 = (t_if[:, hdim:] * ch + ch) + (t_if[:, :hdim] * gh + gh)
        th = 0.5 * jnp.tanh(c2)
        h = t_o * th + th
        return h, c2

    chunked = Lc >= 7 + (4 * H // 256 if H % 256 == 0 else 1)

    def body(wseq_ref, cseq_ref, wemb_hbm, wih2_hbm, whh2_hbm, wout_hbm,
             ce_ref, wih1_ref, whh1_ref, bih1_ref, bhh1_ref,
             ch0_ref, cc0_ref, bih2_ref, bhh2_ref, h0_ref, c0_ref, bout_ref,
             out_ref,
             wemb_scr, wih2_scr, whh2_scr, wout_scr,
             xg2_scr, whh2t_scr, ho_scr, gsem, wsem):
        for w in range(W):
            pltpu.make_async_copy(wemb_hbm.at[wseq_ref[w]],
                                  wemb_scr.at[w], gsem).start()
        pltpu.make_async_copy(wih2_hbm, wih2_scr, wsem).start()
        pltpu.make_async_copy(whh2_hbm, whh2_scr, wsem).start()
        pltpu.make_async_copy(wout_hbm, wout_scr, wsem).start()

        wih1t = wih1_ref[...].T.astype(bf)
        whh1t = _halve_ifo(whh1_ref[...].T, Hc).astype(bf)
        b1 = bih1_ref[...] + bhh1_ref[...]
        ce_projb = _halve_ifo(
            jnp.dot(ce_ref[...].astype(bf), wih1t,
                    preferred_element_type=jnp.float32) + b1,
            Hc).astype(bf)

        def _whh2t_chunk(k):
            if H % 256:
                whh2t_scr[...] = _halve_ifo(whh2_scr[...].T,
                                            H).astype(bf)
                return
            x = whh2_scr[256 * k:256 * (k + 1), :].T
            if 256 * (k + 1) <= 2 * H or 256 * k >= 3 * H:
                x = x * 0.5
            whh2t_scr[:, 256 * k:256 * (k + 1)] = x.astype(bf)

        def _word_prep(t):
            if t == 0:
                pltpu.make_async_copy(wih2_scr, wih2_scr, wsem).wait()
                pltpu.make_async_copy(whh2_scr, whh2_scr, wsem).wait()
                pltpu.make_async_copy(wout_scr, wout_scr, wsem).wait()
            if t == 1:
                prep["w2wt"] = wih2_scr[:, Hc:].T.astype(bf)
                prep["b2"] = bih2_ref[...] + bhh2_ref[...]
            if t == 2:
                pltpu.make_async_copy(wemb_scr.at[pl.ds(0, W)],
                                      wemb_scr.at[pl.ds(0, W)], gsem).wait()
                xg2_scr[...] = (jnp.dot(wemb_scr[...].astype(bf),
                                        prep["w2wt"],
                                        preferred_element_type=jnp.float32)
                                + prep["b2"])
            if t == 3:
                prep["w2ct"] = wih2_scr[:, :Hc].T.astype(bf)
            if 4 <= t < 4 + n_chunks:
                _whh2t_chunk(t - 4)
            if t == 4 + n_chunks:
                prep["woutt"] = wout_scr[...].T.astype(bf)

        prep = {}
        n_chunks = 4 * H // 256 if H % 256 == 0 else 1
        n_prep = 5 + n_chunks

        h = jnp.broadcast_to(ch0_ref[...], (Wp, Hc))
        c = jnp.broadcast_to(cc0_ref[...], (Wp, Hc))
        iota = jax.lax.broadcasted_iota(jnp.int32, (Wp, Cs), 1)
        for t in range(Lc):
            onehot = (iota == cseq_ref[:, t:t + 1]).astype(bf)
            gates = (jnp.dot(onehot, ce_projb,
                             preferred_element_type=jnp.float32)
                     + jnp.dot(h.astype(bf), whh1t,
                               preferred_element_type=jnp.float32))
            h, c = _cell(gates, c, Hc)
        for t in range(n_prep):
            _word_prep(t)

        xg2_scr[...] = _halve_ifo(
            xg2_scr[...] + jnp.dot(h.astype(bf), prep["w2ct"],
                                   preferred_element_type=jnp.float32), H)

        h2 = h0_ref[...]
        c2 = c0_ref[...]
        for w in range(W):
            gates = xg2_scr[w:w + 1, :] + jnp.dot(
                h2.astype(bf), whh2t_scr[...],
                preferred_element_type=jnp.float32)
            h2, c2 = _cell(gates, c2, H)
            ho_scr[w:w + 1, :] = h2.astype(bf)

        tag = (jnp.dot(ho_scr[...], prep["woutt"],
                       preferred_element_type=jnp.float32) + bout_ref[...])
        m = jnp.max(tag, axis=1, keepdims=True)
        z = tag - m
        lse = jnp.log(jnp.sum(jnp.exp(z), axis=1, keepdims=True))
        out_ref[...] = z - lse

    vmem_inputs = (char_embedding,
                   w_ih1, w_hh1, b_ih1.reshape(1, -1), b_hh1.reshape(1, -1),
                   c_hx.reshape(1, Hc), c_cx.reshape(1, Hc),
                   b_ih2.reshape(1, -1), b_hh2.reshape(1, -1),
                   hx.reshape(1, H), cx.reshape(1, H),
                   b_out.reshape(1, -1))

    def _full(shape):
        nd = len(shape)
        return pl.BlockSpec(shape, lambda i, _nd=nd: (0,) * _nd)

    any_spec = pl.BlockSpec(memory_space=pl.ANY)
    out = pl.pallas_call(
        body,
        out_shape=jax.ShapeDtypeStruct((Wp, T), jnp.float32),
        grid=(1,),
        in_specs=[pl.BlockSpec(memory_space=pltpu.SMEM),
                  _full(cseq.shape),
                  any_spec, any_spec, any_spec, any_spec]
                 + [_full(x.shape) for x in vmem_inputs],
        out_specs=_full((Wp, T)),
        scratch_shapes=[pltpu.VMEM((Wp, Ew), jnp.float32),
                        pltpu.VMEM(w_ih2.shape, jnp.float32),
                        pltpu.VMEM(w_hh2.shape, jnp.float32),
                        pltpu.VMEM(w_out.shape, jnp.float32),
                        pltpu.VMEM((Wp, 4 * H), jnp.float32),
                        pltpu.VMEM((H, 4 * H), bf),
                        pltpu.VMEM((Wp, H), bf),
                        pltpu.SemaphoreType.DMA,
                        pltpu.SemaphoreType.DMA],
        compiler_params=pltpu.CompilerParams(
            dimension_semantics=("arbitrary",)),
    )(w_seq, cseq, word_embedding, w_ih2, w_hh2, w_out, *vmem_inputs)
    if Wp != W:
        out = out[:W]
    return out


# speedup vs baseline: 1.0540x; 1.0186x over previous
import jax
import jax.numpy as jnp
from jax.experimental import pallas as pl
from jax.experimental.pallas import tpu as pltpu


def _sigmoid(x):
    return 0.5 * (jnp.tanh(0.5 * x) + 1.0)


def kernel(w_seq, c_seq, char_embedding, word_embedding,
           w_ih1, w_hh1, b_ih1, b_hh1, c_hx, c_cx,
           w_ih2, w_hh2, b_ih2, b_hh2, hx, cx, w_out, b_out):
    W = int(w_seq.shape[0])
    Lc = int(c_seq.shape[1])
    Hc = int(c_hx.shape[-1])
    Ew = int(word_embedding.shape[1])
    H = int(hx.shape[-1])
    T = int(w_out.shape[0])
    Cs = int(char_embedding.shape[0])
    Wp = ((W + 7) // 8) * 8
    bf = jnp.bfloat16

    cseq = c_seq
    if Wp != W:
        cseq = jnp.pad(cseq, ((0, Wp - W), (0, 0)))

    def body(wseq_ref, cseq_ref, wemb_hbm, wih2_hbm, whh2_hbm, wout_hbm,
             ce_ref, wih1_ref, whh1_ref, bih1_ref, bhh1_ref,
             ch0_ref, cc0_ref, bih2_ref, bhh2_ref, h0_ref, c0_ref, bout_ref,
             out_ref,
             wemb_scr, wih2_scr, whh2_scr, wout_scr, xg2_scr, ho_scr,
             gsem, wsem):
        for w in range(W):
            pltpu.make_async_copy(wemb_hbm.at[wseq_ref[w]],
                                  wemb_scr.at[w], gsem).start()
        pltpu.make_async_copy(wih2_hbm, wih2_scr, wsem).start()
        pltpu.make_async_copy(whh2_hbm, whh2_scr, wsem).start()
        pltpu.make_async_copy(wout_hbm, wout_scr, wsem).start()

        wih1t = wih1_ref[...].T.astype(bf)
        whh1t = whh1_ref[...].T.astype(bf)
        b1 = bih1_ref[...] + bhh1_ref[...]
        ce_projb = (jnp.dot(ce_ref[...].astype(bf), wih1t,
                            preferred_element_type=jnp.float32)
                    + b1).astype(bf)

        h = jnp.broadcast_to(ch0_ref[...], (Wp, Hc))
        c = jnp.broadcast_to(cc0_ref[...], (Wp, Hc))
        iota = jax.lax.broadcasted_iota(jnp.int32, (Wp, Cs), 1)
        for t in range(Lc):
            onehot = (iota == cseq_ref[:, t:t + 1]).astype(bf)
            gates = (jnp.dot(onehot, ce_projb,
                             preferred_element_type=jnp.float32)
                     + jnp.dot(h.astype(bf), whh1t,
                               preferred_element_type=jnp.float32))
            sif = _sigmoid(gates[:, :2 * Hc])
            g = jnp.tanh(gates[:, 2 * Hc:3 * Hc])
            so = _sigmoid(gates[:, 3 * Hc:])
            c = sif[:, Hc:] * c + sif[:, :Hc] * g
            h = so * jnp.tanh(c)

        pltpu.make_async_copy(wemb_scr.at[pl.ds(0, W)],
                              wemb_scr.at[pl.ds(0, W)], gsem).wait()
        pltpu.make_async_copy(wih2_scr, wih2_scr, wsem).wait()
        pltpu.make_async_copy(whh2_scr, whh2_scr, wsem).wait()
        pltpu.make_async_copy(wout_scr, wout_scr, wsem).wait()
        wih2t = wih2_scr[...].T.astype(bf)
        b2 = bih2_ref[...] + bhh2_ref[...]
        x2 = jnp.concatenate(
            [h.astype(bf), wemb_scr[...].astype(bf)], axis=1)
        xg2_scr[...] = (jnp.dot(x2, wih2t,
                                preferred_element_type=jnp.float32) + b2)
        whh2t = whh2_scr[...].T.astype(bf)
        woutt = wout_scr[...].T.astype(bf)

        h2 = h0_ref[...]
        c2 = c0_ref[...]
        for w in range(W):
            gates = xg2_scr[w:w + 1, :] + jnp.dot(
                h2.astype(bf), whh2t, preferred_element_type=jnp.float32)
            sif = _sigmoid(gates[:, :2 * H])
            g = jnp.tanh(gates[:, 2 * H:3 * H])
            so = _sigmoid(gates[:, 3 * H:])
            c2 = sif[:, H:] * c2 + sif[:, :H] * g
            h2 = so * jnp.tanh(c2)
            ho_scr[w:w + 1, :] = h2.astype(bf)

        tag = (jnp.dot(ho_scr[...], woutt,
                       preferred_element_type=jnp.float32) + bout_ref[...])
        m = jnp.max(tag, axis=1, keepdims=True)
        z = tag - m
        lse = jnp.log(jnp.sum(jnp.exp(z), axis=1, keepdims=True))
        out_ref[...] = z - lse

    vmem_inputs = (char_embedding,
                   w_ih1, w_hh1, b_ih1.reshape(1, -1), b_hh1.reshape(1, -1),
                   c_hx.reshape(1, Hc), c_cx.reshape(1, Hc),
                   b_ih2.reshape(1, -1), b_hh2.reshape(1, -1),
                   hx.reshape(1, H), cx.reshape(1, H),
                   b_out.reshape(1, -1))

    def _full(shape):
        nd = len(shape)
        return pl.BlockSpec(shape, lambda i, _nd=nd: (0,) * _nd)

    any_spec = pl.BlockSpec(memory_space=pl.ANY)
    out = pl.pallas_call(
        body,
        out_shape=jax.ShapeDtypeStruct((Wp, T), jnp.float32),
        grid=(1,),
        in_specs=[pl.BlockSpec(memory_space=pltpu.SMEM),
                  _full(cseq.shape),
                  any_spec, any_spec, any_spec, any_spec]
                 + [_full(x.shape) for x in vmem_inputs],
        out_specs=_full((Wp, T)),
        scratch_shapes=[pltpu.VMEM((Wp, Ew), jnp.float32),
                        pltpu.VMEM(w_ih2.shape, jnp.float32),
                        pltpu.VMEM(w_hh2.shape, jnp.float32),
                        pltpu.VMEM(w_out.shape, jnp.float32),
                        pltpu.VMEM((Wp, 4 * H), jnp.float32),
                        pltpu.VMEM((Wp, H), bf),
                        pltpu.SemaphoreType.DMA,
                        pltpu.SemaphoreType.DMA],
        compiler_params=pltpu.CompilerParams(
            dimension_semantics=("arbitrary",)),
    )(w_seq, cseq, word_embedding, w_ih2, w_hh2, w_out, *vmem_inputs)
    if Wp != W:
        out = out[:W]
    return out
